# baseline (speedup 1.0000x reference)
import jax
import jax.numpy as jnp
from jax.experimental import pallas as pl
from jax.experimental.pallas import tpu as pltpu

_B, _C, _H, _W = 64, 768, 24, 24
_HW = _H * _W
_NSLOT = 2


def _body(t_ref, s_ref, emb_ref, x_hbm, out_hbm, mask_hbm,
          xbuf, obuf, mbuf, sin, sout, smask):
    del t_ref
    m2 = jax.nn.sigmoid(s_ref[0, 0] * emb_ref[0, 0, :])[:, None]
    mask_slab = jnp.broadcast_to(m2, (_C, _HW))
    mbuf[0] = mask_slab
    mbuf[1] = mask_slab

    def cp_in(b, slot):
        return pltpu.make_async_copy(x_hbm.at[b], xbuf.at[slot], sin.at[slot])

    def cp_out(b, slot):
        return pltpu.make_async_copy(obuf.at[slot], out_hbm.at[b], sout.at[slot])

    def cp_mask(b, slot):
        return pltpu.make_async_copy(mbuf.at[slot], mask_hbm.at[b], smask.at[slot])

    cp_in(0, 0).start()

    def step(b, carry):
        slot = jax.lax.rem(b, _NSLOT)

        @pl.when(b + 1 < _B)
        def _():
            cp_in(b + 1, jax.lax.rem(b + 1, _NSLOT)).start()

        @pl.when(b >= _NSLOT)
        def _():
            cp_out(b - _NSLOT, slot).wait()
            cp_mask(b - _NSLOT, slot).wait()

        cp_in(b, slot).wait()
        obuf[pl.ds(slot, 1)] = xbuf[pl.ds(slot, 1)] * m2[None]
        cp_out(b, slot).start()
        cp_mask(b, slot).start()
        return carry

    jax.lax.fori_loop(0, _B, step, 0)

    cp_out(_B - 2, 0).wait()
    cp_mask(_B - 2, 0).wait()
    cp_out(_B - 1, 1).wait()
    cp_mask(_B - 1, 1).wait()


def kernel(t, x, s, embedding):
    x3 = x.reshape(_B, _C, _HW)
    s2 = s.reshape(1, 1)
    t32 = t.astype(jnp.int32)

    out, mask = pl.pallas_call(
        _body,
        grid_spec=pltpu.PrefetchScalarGridSpec(
            num_scalar_prefetch=1,
            grid=(1,),
            in_specs=[
                pl.BlockSpec((1, 1), lambda i, t_ref: (0, 0)),
                pl.BlockSpec((1, 1, _C), lambda i, t_ref: (t_ref[0], 0, 0)),
                pl.BlockSpec(memory_space=pl.ANY),
            ],
            out_specs=[
                pl.BlockSpec(memory_space=pl.ANY),
                pl.BlockSpec(memory_space=pl.ANY),
            ],
            scratch_shapes=[
                pltpu.VMEM((_NSLOT, _C, _HW), jnp.float32),
                pltpu.VMEM((_NSLOT, _C, _HW), jnp.float32),
                pltpu.VMEM((_NSLOT, _C, _HW), jnp.float32),
                pltpu.SemaphoreType.DMA((_NSLOT,)),
                pltpu.SemaphoreType.DMA((_NSLOT,)),
                pltpu.SemaphoreType.DMA((_NSLOT,)),
            ],
        ),
        out_shape=[
            jax.ShapeDtypeStruct((_B, _C, _HW), jnp.float32),
            jax.ShapeDtypeStruct((_B, _C, _HW), jnp.float32),
        ],
    )(t32, s2, embedding.reshape(100, 1, _C), x3)

    return out.reshape(x.shape), mask.reshape(x.shape)


# speedup vs baseline: 1.0060x; 1.0060x over previous
import jax
import jax.numpy as jnp
from jax.experimental import pallas as pl
from jax.experimental.pallas import tpu as pltpu

_B, _C, _H, _W = 64, 768, 24, 24
_HW = _H * _W
_NS = 8


def _body(t_ref, s_ref, emb_ref, x_hbm, out_hbm, mask_hbm,
          xbuf, obuf, mbuf, sin, sout, smask):
    del t_ref
    m2 = jax.nn.sigmoid(s_ref[0, 0] * emb_ref[0, 0, :])[:, None]
    mbuf[...] = jnp.broadcast_to(m2, (_C, _HW))

    def cp_in(b, slot):
        return pltpu.make_async_copy(x_hbm.at[b], xbuf.at[slot], sin.at[slot])

    def cp_out(b, slot):
        return pltpu.make_async_copy(obuf.at[slot], out_hbm.at[b], sout.at[slot])

    def cp_mask(b, slot):
        return pltpu.make_async_copy(mbuf, mask_hbm.at[b], smask.at[slot])

    for i in range(_NS):
        cp_in(i, i).start()
        cp_mask(i, i).start()

    def step(b, carry):
        slot = jax.lax.rem(b, _NS)

        @pl.when(b >= _NS)
        def _():
            cp_out(b - _NS, slot).wait()
            cp_mask(b - _NS, slot).wait()
            cp_mask(b, slot).start()

        cp_in(b, slot).wait()
        obuf[pl.ds(slot, 1)] = xbuf[pl.ds(slot, 1)] * m2[None]
        cp_out(b, slot).start()

        @pl.when(b + _NS < _B)
        def _():
            cp_in(b + _NS, slot).start()

        return carry

    jax.lax.fori_loop(0, _B, step, 0, unroll=2)

    for i in range(_B - _NS, _B):
        cp_out(i, i % _NS).wait()
        cp_mask(i, i % _NS).wait()


def kernel(t, x, s, embedding):
    x3 = x.reshape(_B, _C, _HW)
    s2 = s.reshape(1, 1)
    t32 = t.astype(jnp.int32)

    out, mask = pl.pallas_call(
        _body,
        grid_spec=pltpu.PrefetchScalarGridSpec(
            num_scalar_prefetch=1,
            grid=(1,),
            in_specs=[
                pl.BlockSpec((1, 1), lambda i, t_ref: (0, 0)),
                pl.BlockSpec((1, 1, _C), lambda i, t_ref: (t_ref[0], 0, 0)),
                pl.BlockSpec(memory_space=pl.ANY),
            ],
            out_specs=[
                pl.BlockSpec(memory_space=pl.ANY),
                pl.BlockSpec(memory_space=pl.ANY),
            ],
            scratch_shapes=[
                pltpu.VMEM((_NS, _C, _HW), jnp.float32),
                pltpu.VMEM((_NS, _C, _HW), jnp.float32),
                pltpu.VMEM((_C, _HW), jnp.float32),
                pltpu.SemaphoreType.DMA((_NS,)),
                pltpu.SemaphoreType.DMA((_NS,)),
                pltpu.SemaphoreType.DMA((_NS,)),
            ],
        ),
        out_shape=[
            jax.ShapeDtypeStruct((_B, _C, _HW), jnp.float32),
            jax.ShapeDtypeStruct((_B, _C, _HW), jnp.float32),
        ],
    )(t32, s2, embedding.reshape(100, 1, _C), x3)

    return out.reshape(x.shape), mask.reshape(x.shape)


# speedup vs baseline: 3.4377x; 3.4172x over previous
import jax
import jax.numpy as jnp
from jax.experimental import pallas as pl
from jax.experimental.pallas import tpu as pltpu
from jax.experimental.pallas import tpu_sc as plsc

_B, _C, _H, _W = 64, 768, 24, 24
_HW = _H * _W
_BB = 2
_NQ = 4
_RPW = _HW // _NQ
_NG = 8
_BPG = _B // _NG


def _mul_body(t_ref, s_ref, emb_ref, x_ref, out_ref):
    del t_ref
    m = jax.nn.sigmoid(s_ref[0, 0] * emb_ref[0, 0, :])
    out_ref[...] = x_ref[...] * m[None, None, :]


def _sc_mask_body(t_hbm, s_hbm, emb_hbm, mask_hbm, tv, sv, row, slab, sems):
    wid = jax.lax.axis_index("c") * 16 + jax.lax.axis_index("s")
    q = jax.lax.rem(wid, _NQ)
    g = jax.lax.div(wid, _NQ)
    pltpu.sync_copy(t_hbm, tv.at[pl.ds(0, 1)])
    pltpu.sync_copy(s_hbm, sv.at[pl.ds(0, 1)])
    ts = tv[...][0]
    sval = sv[...][0]
    pltpu.sync_copy(emb_hbm.at[pl.ds(ts, 1)], row)

    @pl.loop(0, _C, step=16)
    def _(j):
        z = row[0, 0, pl.ds(j, 16)]
        slab[0, pl.ds(j, 16)] = 1.0 / (1.0 + jnp.exp(-sval * z))

    @pl.loop(1, _RPW)
    def _(r):
        @pl.loop(0, _C, step=16)
        def _(j):
            slab[r, pl.ds(j, 16)] = slab[0, pl.ds(j, 16)]

    base = q * _RPW

    def cp(i, slot):
        return pltpu.make_async_copy(
            slab, mask_hbm.at[g * _BPG + i, pl.ds(base, _RPW)], sems.at[slot])

    @pl.loop(0, _BPG)
    def _(i):
        cp(i, i).start()

    @pl.loop(0, _BPG)
    def _(i):
        cp(i, i).wait()


def kernel(t, x, s, embedding):
    xt = jnp.transpose(x, (0, 2, 3, 1)).reshape(_B, _HW, _C)
    s2 = s.reshape(1, 1)
    t32 = t.astype(jnp.int32)

    sc_mask = pl.kernel(
        _sc_mask_body,
        out_type=jax.ShapeDtypeStruct((_B, _HW, _C), jnp.float32),
        mesh=plsc.VectorSubcoreMesh(core_axis_name="c", subcore_axis_name="s"),
        scratch_types=[
            pltpu.VMEM((16,), jnp.int32),
            pltpu.VMEM((16,), jnp.float32),
            pltpu.VMEM((1, 1, _C), jnp.float32),
            pltpu.VMEM((_RPW, _C), jnp.float32),
            pltpu.SemaphoreType.DMA((_BPG,)),
        ],
    )
    mask = sc_mask(t32, s, embedding.reshape(100, 1, _C))

    out = pl.pallas_call(
        _mul_body,
        grid_spec=pltpu.PrefetchScalarGridSpec(
            num_scalar_prefetch=1,
            grid=(_B // _BB,),
            in_specs=[
                pl.BlockSpec((1, 1), lambda b, t_ref: (0, 0)),
                pl.BlockSpec((1, 1, _C), lambda b, t_ref: (t_ref[0], 0, 0)),
                pl.BlockSpec((_BB, _HW, _C), lambda b, t_ref: (b, 0, 0)),
            ],
            out_specs=pl.BlockSpec((_BB, _HW, _C), lambda b, t_ref: (b, 0, 0)),
        ),
        out_shape=jax.ShapeDtypeStruct((_B, _HW, _C), jnp.float32),
    )(t32, s2, embedding.reshape(100, 1, _C), xt)

    out4 = jnp.transpose(out.reshape(_B, _H, _W, _C), (0, 3, 1, 2))
    mask4 = jnp.transpose(mask.reshape(_B, _H, _W, _C), (0, 3, 1, 2))
    return out4, mask4


# speedup vs baseline: 4.1564x; 1.2091x over previous
import jax
import jax.numpy as jnp
from jax.experimental import pallas as pl
from jax.experimental.pallas import tpu as pltpu

_B, _C, _H, _W = 64, 768, 24, 24
_HW = _H * _W
_BB = 4


def _body(t_ref, s_ref, emb_ref, x_ref, out_ref, mask_ref):
    del t_ref
    m = jax.nn.sigmoid(s_ref[0, 0] * emb_ref[0, 0, :])
    mrow = m[None, None, :]
    out_ref[...] = x_ref[...] * mrow
    mask_ref[...] = jnp.broadcast_to(mrow, (_BB, _HW, _C))


def kernel(t, x, s, embedding):
    xt = jnp.transpose(x, (0, 2, 3, 1)).reshape(_B, _HW, _C)
    s2 = s.reshape(1, 1)
    t32 = t.astype(jnp.int32)

    out, mask = pl.pallas_call(
        _body,
        grid_spec=pltpu.PrefetchScalarGridSpec(
            num_scalar_prefetch=1,
            grid=(_B // _BB,),
            in_specs=[
                pl.BlockSpec((1, 1), lambda b, t_ref: (0, 0)),
                pl.BlockSpec((1, 1, _C), lambda b, t_ref: (t_ref[0], 0, 0)),
                pl.BlockSpec((_BB, _HW, _C), lambda b, t_ref: (b, 0, 0)),
            ],
            out_specs=[
                pl.BlockSpec((_BB, _HW, _C), lambda b, t_ref: (b, 0, 0)),
                pl.BlockSpec((_BB, _HW, _C), lambda b, t_ref: (b, 0, 0)),
            ],
        ),
        out_shape=[
            jax.ShapeDtypeStruct((_B, _HW, _C), jnp.float32),
            jax.ShapeDtypeStruct((_B, _HW, _C), jnp.float32),
        ],
    )(t32, s2, embedding.reshape(100, 1, _C), xt)

    out4 = jnp.transpose(out.reshape(_B, _H, _W, _C), (0, 3, 1, 2))
    mask4 = jnp.transpose(mask.reshape(_B, _H, _W, _C), (0, 3, 1, 2))
    return out4, mask4
